# baseline (speedup 1.0000x reference)
import functools

import jax
import jax.numpy as jnp
from jax import lax
from jax.experimental import pallas as pl
from jax.experimental.pallas import tpu as pltpu
from jax.experimental.pallas import tpu_sc as plsc

N = 10000
S = 32
D = 128
DOUT = 128
LANES = 16
NC = 2
NS = 16
NW = NC * NS

CH = 8
CHUNK_ROWS = CH * S
NCH = N // CH
ITERS = (NCH + NW - 1) // NW


def _sc_agg_body(src_neg_hbm, dst_neg_hbm, src_sum_hbm, dst_sum_hbm,
                 sbuf, dbuf, sout, dout):
    wid = lax.axis_index("s") * NC + lax.axis_index("c")

    def accumulate(buf, obuf):
        for n in range(CH):
            def body(s, accs):
                row = n * S + s
                return tuple(accs[j] + buf[row, pl.ds(j * LANES, LANES)]
                             for j in range(D // LANES))
            init = tuple(jnp.zeros((LANES,), jnp.float32)
                         for _ in range(D // LANES))
            accs = lax.fori_loop(0, S, body, init, unroll=False)
            for j in range(D // LANES):
                obuf[n, pl.ds(j * LANES, LANES)] = accs[j]

    def one_chunk(c):
        row0 = c * CHUNK_ROWS
        pltpu.sync_copy(src_neg_hbm.at[pl.ds(row0, CHUNK_ROWS)], sbuf)
        accumulate(sbuf, sout)
        pltpu.sync_copy(sout, src_sum_hbm.at[pl.ds(c * CH, CH)])
        pltpu.sync_copy(dst_neg_hbm.at[pl.ds(row0, CHUNK_ROWS)], dbuf)
        accumulate(dbuf, dout)
        pltpu.sync_copy(dout, dst_sum_hbm.at[pl.ds(c * CH, CH)])

    def outer(it, _):
        c = wid + it * NW

        @pl.when(c < NCH)
        def _():
            one_chunk(c)
        return 0

    lax.fori_loop(0, ITERS, outer, 0, unroll=False)


def _sc_aggregate(src_neg, dst_neg):
    mesh = plsc.VectorSubcoreMesh(core_axis_name="c", subcore_axis_name="s")
    f = pl.kernel(
        _sc_agg_body,
        out_type=(jax.ShapeDtypeStruct((N, D), jnp.float32),
                  jax.ShapeDtypeStruct((N, D), jnp.float32)),
        mesh=mesh,
        scratch_types=[
            pltpu.VMEM((CHUNK_ROWS, D), jnp.float32),
            pltpu.VMEM((CHUNK_ROWS, D), jnp.float32),
            pltpu.VMEM((CH, D), jnp.float32),
            pltpu.VMEM((CH, D), jnp.float32),
        ],
    )
    return f(src_neg, dst_neg)


def _tc_body(src_ref, ssum_ref, dst_ref, dsum_ref, w_ref,
             osrc_ref, odst_ref):
    w = w_ref[...]
    inv = jnp.float32(1.0 / S)
    xs = jnp.concatenate([src_ref[...], ssum_ref[...] * inv], axis=1)
    xd = jnp.concatenate([dst_ref[...], dsum_ref[...] * inv], axis=1)
    ys = jax.lax.dot_general(xs, w, (((1,), (0,)), ((), ())),
                             preferred_element_type=jnp.float32,
                             precision=jax.lax.Precision.HIGHEST)
    yd = jax.lax.dot_general(xd, w, (((1,), (0,)), ((), ())),
                             preferred_element_type=jnp.float32,
                             precision=jax.lax.Precision.HIGHEST)
    osrc_ref[...] = jnp.maximum(ys, 0.0)
    odst_ref[...] = jnp.maximum(yd, 0.0)


def _tc_matmul(src, src_sum, dst, dst_sum, w):
    B = 400
    grid = (N // B,)
    row_spec = pl.BlockSpec((B, D), lambda i: (i, 0))
    w_spec = pl.BlockSpec((2 * D, DOUT), lambda i: (0, 0))
    out_spec = pl.BlockSpec((B, DOUT), lambda i: (i, 0))
    return pl.pallas_call(
        _tc_body,
        grid=grid,
        in_specs=[row_spec, row_spec, row_spec, row_spec, w_spec],
        out_specs=[out_spec, out_spec],
        out_shape=(jax.ShapeDtypeStruct((N, DOUT), jnp.float32),
                   jax.ShapeDtypeStruct((N, DOUT), jnp.float32)),
    )(src, src_sum, dst, dst_sum, w)


@jax.jit
def kernel(src, src_neg, dst, dst_neg, w):
    src_sum, dst_sum = _sc_aggregate(src_neg, dst_neg)
    out_src, out_dst = _tc_matmul(src, src_sum, dst, dst_sum, w)
    return (out_src, out_dst)


# speedup vs baseline: 1.3584x; 1.3584x over previous
import jax
import jax.numpy as jnp
from jax import lax
from jax.experimental import pallas as pl
from jax.experimental.pallas import tpu as pltpu
from jax.experimental.pallas import tpu_sc as plsc

N = 10000
S = 32
D = 128
DOUT = 128
LANES = 16
NJ = D // LANES
NC = 2
NS = 16
NW = NC * NS

CH = 8
CHUNK_ROWS = CH * S
NCH = N // CH
TOT = 2 * NCH
SLOTS = 80


def _sc_agg_body(src_neg_hbm, dst_neg_hbm, src_sum_hbm, dst_sum_hbm,
                 buf0, buf1, ob0, ob1, si0, si1, so0, so1):
    wid = lax.axis_index("s") * NC + lax.axis_index("c")
    bufs, obufs = (buf0, buf1), (ob0, ob1)
    sins, souts = (si0, si1), (so0, so1)

    def item_of(j):
        return jnp.minimum(wid + j * NW, TOT - 1)

    def start_in(j, b):
        k = item_of(j)

        @pl.when(k < NCH)
        def _():
            pltpu.async_copy(
                src_neg_hbm.at[pl.ds(k * CHUNK_ROWS, CHUNK_ROWS)],
                bufs[b], sins[b])

        @pl.when(k >= NCH)
        def _():
            pltpu.async_copy(
                dst_neg_hbm.at[pl.ds((k - NCH) * CHUNK_ROWS, CHUNK_ROWS)],
                bufs[b], sins[b])

    def wait_in(b):
        pltpu.make_async_copy(
            src_neg_hbm.at[pl.ds(0, CHUNK_ROWS)], bufs[b], sins[b]).wait()

    def start_out(j, b):
        k = item_of(j)

        @pl.when(k < NCH)
        def _():
            pltpu.async_copy(obufs[b], src_sum_hbm.at[pl.ds(k * CH, CH)],
                             souts[b])

        @pl.when(k >= NCH)
        def _():
            pltpu.async_copy(obufs[b],
                             dst_sum_hbm.at[pl.ds((k - NCH) * CH, CH)],
                             souts[b])

    def wait_out(b):
        pltpu.make_async_copy(
            obufs[b], src_sum_hbm.at[pl.ds(0, CH)], souts[b]).wait()

    def compute(b):
        buf, obuf = bufs[b], obufs[b]

        def node_body(n, _):
            base = n * S
            accs = [buf[base, pl.ds(j * LANES, LANES)] for j in range(NJ)]
            for s in range(1, S):
                for j in range(NJ):
                    accs[j] = accs[j] + buf[base + s, pl.ds(j * LANES, LANES)]
            for j in range(NJ):
                obuf[n, pl.ds(j * LANES, LANES)] = accs[j]
            return 0

        lax.fori_loop(0, CH, node_body, 0, unroll=False)

    start_in(0, 0)

    def outer(jj, _):
        for b in range(2):
            j = jj * 2 + b

            @pl.when(j + 1 < SLOTS)
            def _():
                start_in(j + 1, 1 - b)

            wait_in(b)

            @pl.when(jj >= 1)
            def _():
                wait_out(b)

            compute(b)
            start_out(j, b)
        return 0

    lax.fori_loop(0, SLOTS // 2, outer, 0, unroll=False)
    wait_out(0)
    wait_out(1)


def _sc_aggregate(src_neg, dst_neg):
    mesh = plsc.VectorSubcoreMesh(core_axis_name="c", subcore_axis_name="s")
    f = pl.kernel(
        _sc_agg_body,
        out_type=(jax.ShapeDtypeStruct((N, D), jnp.float32),
                  jax.ShapeDtypeStruct((N, D), jnp.float32)),
        mesh=mesh,
        scratch_types=[
            pltpu.VMEM((CHUNK_ROWS, D), jnp.float32),
            pltpu.VMEM((CHUNK_ROWS, D), jnp.float32),
            pltpu.VMEM((CH, D), jnp.float32),
            pltpu.VMEM((CH, D), jnp.float32),
            pltpu.SemaphoreType.DMA,
            pltpu.SemaphoreType.DMA,
            pltpu.SemaphoreType.DMA,
            pltpu.SemaphoreType.DMA,
        ],
    )
    return f(src_neg, dst_neg)


def _tc_body(src_ref, ssum_ref, dst_ref, dsum_ref, w_ref,
             osrc_ref, odst_ref):
    w = w_ref[...]
    inv = jnp.float32(1.0 / S)
    xs = jnp.concatenate([src_ref[...], ssum_ref[...] * inv], axis=1)
    xd = jnp.concatenate([dst_ref[...], dsum_ref[...] * inv], axis=1)
    ys = jax.lax.dot_general(xs, w, (((1,), (0,)), ((), ())),
                             preferred_element_type=jnp.float32,
                             precision=jax.lax.Precision.HIGHEST)
    yd = jax.lax.dot_general(xd, w, (((1,), (0,)), ((), ())),
                             preferred_element_type=jnp.float32,
                             precision=jax.lax.Precision.HIGHEST)
    osrc_ref[...] = jnp.maximum(ys, 0.0)
    odst_ref[...] = jnp.maximum(yd, 0.0)


def _tc_matmul(src, src_sum, dst, dst_sum, w):
    B = 400
    grid = (N // B,)
    row_spec = pl.BlockSpec((B, D), lambda i: (i, 0))
    w_spec = pl.BlockSpec((2 * D, DOUT), lambda i: (0, 0))
    out_spec = pl.BlockSpec((B, DOUT), lambda i: (i, 0))
    return pl.pallas_call(
        _tc_body,
        grid=grid,
        in_specs=[row_spec, row_spec, row_spec, row_spec, w_spec],
        out_specs=[out_spec, out_spec],
        out_shape=(jax.ShapeDtypeStruct((N, DOUT), jnp.float32),
                   jax.ShapeDtypeStruct((N, DOUT), jnp.float32)),
    )(src, src_sum, dst, dst_sum, w)


@jax.jit
def kernel(src, src_neg, dst, dst_neg, w):
    src_sum, dst_sum = _sc_aggregate(src_neg, dst_neg)
    out_src, out_dst = _tc_matmul(src, src_sum, dst, dst_sum, w)
    return (out_src, out_dst)


# speedup vs baseline: 2.1294x; 1.5676x over previous
import jax
import jax.numpy as jnp
from jax import lax
from jax.experimental import pallas as pl
from jax.experimental.pallas import tpu as pltpu
from jax.experimental.pallas import tpu_sc as plsc

N = 10000
S = 32
D = 128
DOUT = 128
LANES = 16
NJ = D // LANES
NC = 2
NS = 16
NW = NC * NS

NSC = 4000
NTC = N - NSC
CH = 8
CHUNK_ROWS = CH * S
NCH = NSC // CH
TOT = 2 * NCH
SLOTS = 2 * ((TOT + 2 * NW - 1) // (2 * NW))


def _sc_agg_body(src_neg_hbm, dst_neg_hbm, src_sum_hbm, dst_sum_hbm,
                 buf0, buf1, ob0, ob1, si0, si1, so0, so1):
    wid = lax.axis_index("s") * NC + lax.axis_index("c")
    bufs, obufs = (buf0, buf1), (ob0, ob1)
    sins, souts = (si0, si1), (so0, so1)

    def item_of(j):
        return jnp.minimum(wid + j * NW, TOT - 1)

    def start_in(j, b):
        k = item_of(j)

        @pl.when(k < NCH)
        def _():
            pltpu.async_copy(
                src_neg_hbm.at[pl.ds(k * CHUNK_ROWS, CHUNK_ROWS)],
                bufs[b], sins[b])

        @pl.when(k >= NCH)
        def _():
            pltpu.async_copy(
                dst_neg_hbm.at[pl.ds((k - NCH) * CHUNK_ROWS, CHUNK_ROWS)],
                bufs[b], sins[b])

    def wait_in(b):
        pltpu.make_async_copy(
            src_neg_hbm.at[pl.ds(0, CHUNK_ROWS)], bufs[b], sins[b]).wait()

    def start_out(j, b):
        k = item_of(j)

        @pl.when(k < NCH)
        def _():
            pltpu.async_copy(obufs[b], src_sum_hbm.at[pl.ds(k * CH, CH)],
                             souts[b])

        @pl.when(k >= NCH)
        def _():
            pltpu.async_copy(obufs[b],
                             dst_sum_hbm.at[pl.ds((k - NCH) * CH, CH)],
                             souts[b])

    def wait_out(b):
        pltpu.make_async_copy(
            obufs[b], src_sum_hbm.at[pl.ds(0, CH)], souts[b]).wait()

    def compute(b):
        buf, obuf = bufs[b], obufs[b]

        def node_body(n, _):
            base = n * S
            accs = [buf[base, pl.ds(j * LANES, LANES)] for j in range(NJ)]
            for s in range(1, S):
                for j in range(NJ):
                    accs[j] = accs[j] + buf[base + s, pl.ds(j * LANES, LANES)]
            for j in range(NJ):
                obuf[n, pl.ds(j * LANES, LANES)] = accs[j]
            return 0

        lax.fori_loop(0, CH, node_body, 0, unroll=False)

    start_in(0, 0)

    def outer(jj, _):
        for b in range(2):
            j = jj * 2 + b

            @pl.when(j + 1 < SLOTS)
            def _():
                start_in(j + 1, 1 - b)

            wait_in(b)

            @pl.when(jj >= 1)
            def _():
                wait_out(b)

            compute(b)
            start_out(j, b)
        return 0

    lax.fori_loop(0, SLOTS // 2, outer, 0, unroll=False)
    wait_out(0)
    wait_out(1)


def _sc_aggregate(src_neg, dst_neg):
    mesh = plsc.VectorSubcoreMesh(core_axis_name="c", subcore_axis_name="s")
    f = pl.kernel(
        _sc_agg_body,
        out_type=(jax.ShapeDtypeStruct((NSC, D), jnp.float32),
                  jax.ShapeDtypeStruct((NSC, D), jnp.float32)),
        mesh=mesh,
        scratch_types=[
            pltpu.VMEM((CHUNK_ROWS, D), jnp.float32),
            pltpu.VMEM((CHUNK_ROWS, D), jnp.float32),
            pltpu.VMEM((CH, D), jnp.float32),
            pltpu.VMEM((CH, D), jnp.float32),
            pltpu.SemaphoreType.DMA,
            pltpu.SemaphoreType.DMA,
            pltpu.SemaphoreType.DMA,
            pltpu.SemaphoreType.DMA,
        ],
    )
    return f(src_neg, dst_neg)


def _dot(x, w):
    return jax.lax.dot_general(x, w, (((1,), (0,)), ((), ())),
                               preferred_element_type=jnp.float32,
                               precision=jax.lax.Precision.HIGHEST)


def _tc_fused_body(src_ref, sneg_ref, dst_ref, dneg_ref, w_ref,
                   osrc_ref, odst_ref):
    B = src_ref.shape[0]
    w = w_ref[...]
    sagg = jnp.mean(jnp.reshape(sneg_ref[...], (B, S, D)), axis=1)
    dagg = jnp.mean(jnp.reshape(dneg_ref[...], (B, S, D)), axis=1)
    xs = jnp.concatenate([src_ref[...], sagg], axis=1)
    xd = jnp.concatenate([dst_ref[...], dagg], axis=1)
    osrc_ref[...] = jnp.maximum(_dot(xs, w), 0.0)
    odst_ref[...] = jnp.maximum(_dot(xd, w), 0.0)


def _tc_fused(src, src_neg, dst, dst_neg, w):
    B = 400
    nb = NTC // B
    grid = (nb,)
    row_spec = pl.BlockSpec((B, D), lambda i: (NSC // B + i, 0))
    neg_spec = pl.BlockSpec((B * S, D), lambda i: (NSC // B + i, 0))
    w_spec = pl.BlockSpec((2 * D, DOUT), lambda i: (0, 0))
    out_spec = pl.BlockSpec((B, DOUT), lambda i: (i, 0))
    return pl.pallas_call(
        _tc_fused_body,
        grid=grid,
        in_specs=[row_spec, neg_spec, row_spec, neg_spec, w_spec],
        out_specs=[out_spec, out_spec],
        out_shape=(jax.ShapeDtypeStruct((NTC, DOUT), jnp.float32),
                   jax.ShapeDtypeStruct((NTC, DOUT), jnp.float32)),
    )(src, src_neg, dst, dst_neg, w)


def _tc_head_body(src_ref, ssum_ref, dst_ref, dsum_ref, w_ref,
                  osrc_ref, odst_ref):
    w = w_ref[...]
    inv = jnp.float32(1.0 / S)
    xs = jnp.concatenate([src_ref[...], ssum_ref[...] * inv], axis=1)
    xd = jnp.concatenate([dst_ref[...], dsum_ref[...] * inv], axis=1)
    osrc_ref[...] = jnp.maximum(_dot(xs, w), 0.0)
    odst_ref[...] = jnp.maximum(_dot(xd, w), 0.0)


def _tc_head(src, src_sum, dst, dst_sum, w):
    B = 400
    grid = (NSC // B,)
    row_spec = pl.BlockSpec((B, D), lambda i: (i, 0))
    w_spec = pl.BlockSpec((2 * D, DOUT), lambda i: (0, 0))
    out_spec = pl.BlockSpec((B, DOUT), lambda i: (i, 0))
    return pl.pallas_call(
        _tc_head_body,
        grid=grid,
        in_specs=[row_spec, row_spec, row_spec, row_spec, w_spec],
        out_specs=[out_spec, out_spec],
        out_shape=(jax.ShapeDtypeStruct((NSC, DOUT), jnp.float32),
                   jax.ShapeDtypeStruct((NSC, DOUT), jnp.float32)),
    )(src, src_sum, dst, dst_sum, w)


@jax.jit
def kernel(src, src_neg, dst, dst_neg, w):
    src_sum, dst_sum = _sc_aggregate(src_neg, dst_neg)
    tail_src, tail_dst = _tc_fused(src, src_neg, dst, dst_neg, w)
    head_src, head_dst = _tc_head(src, src_sum, dst, dst_sum, w)
    out_src = jnp.concatenate([head_src, tail_src], axis=0)
    out_dst = jnp.concatenate([head_dst, tail_dst], axis=0)
    return (out_src, out_dst)


# speedup vs baseline: 2.2426x; 1.0532x over previous
import jax
import jax.numpy as jnp
from jax import lax
from jax.experimental import pallas as pl
from jax.experimental.pallas import tpu as pltpu
from jax.experimental.pallas import tpu_sc as plsc

N = 10000
S = 32
D = 128
DOUT = 128
LANES = 16
NJ = D // LANES
NC = 2
NS = 16
NW = NC * NS

NSC = 3200
NTC = N - NSC
CH = 8
CHUNK_ROWS = CH * S
NCH = NSC // CH
TOT = 2 * NCH
SLOTS = 2 * ((TOT + 2 * NW - 1) // (2 * NW))


def _sc_agg_body(src_neg_hbm, dst_neg_hbm, src_sum_hbm, dst_sum_hbm,
                 buf0, buf1, ob0, ob1, si0, si1, so0, so1):
    wid = lax.axis_index("s") * NC + lax.axis_index("c")
    bufs, obufs = (buf0, buf1), (ob0, ob1)
    sins, souts = (si0, si1), (so0, so1)

    def item_of(j):
        return jnp.minimum(wid + j * NW, TOT - 1)

    def start_in(j, b):
        k = item_of(j)

        @pl.when(k < NCH)
        def _():
            pltpu.async_copy(
                src_neg_hbm.at[pl.ds(k * CHUNK_ROWS, CHUNK_ROWS)],
                bufs[b], sins[b])

        @pl.when(k >= NCH)
        def _():
            pltpu.async_copy(
                dst_neg_hbm.at[pl.ds((k - NCH) * CHUNK_ROWS, CHUNK_ROWS)],
                bufs[b], sins[b])

    def wait_in(b):
        pltpu.make_async_copy(
            src_neg_hbm.at[pl.ds(0, CHUNK_ROWS)], bufs[b], sins[b]).wait()

    def start_out(j, b):
        k = item_of(j)

        @pl.when(k < NCH)
        def _():
            pltpu.async_copy(obufs[b], src_sum_hbm.at[pl.ds(k * CH, CH)],
                             souts[b])

        @pl.when(k >= NCH)
        def _():
            pltpu.async_copy(obufs[b],
                             dst_sum_hbm.at[pl.ds((k - NCH) * CH, CH)],
                             souts[b])

    def wait_out(b):
        pltpu.make_async_copy(
            obufs[b], src_sum_hbm.at[pl.ds(0, CH)], souts[b]).wait()

    def compute(b):
        buf, obuf = bufs[b], obufs[b]

        def node_body(n, _):
            base = n * S
            accs = [buf[base, pl.ds(j * LANES, LANES)] for j in range(NJ)]
            for s in range(1, S):
                for j in range(NJ):
                    accs[j] = accs[j] + buf[base + s, pl.ds(j * LANES, LANES)]
            for j in range(NJ):
                obuf[n, pl.ds(j * LANES, LANES)] = accs[j]
            return 0

        lax.fori_loop(0, CH, node_body, 0, unroll=False)

    start_in(0, 0)

    def outer(jj, _):
        for b in range(2):
            j = jj * 2 + b

            @pl.when(j + 1 < SLOTS)
            def _():
                start_in(j + 1, 1 - b)

            wait_in(b)

            @pl.when(jj >= 1)
            def _():
                wait_out(b)

            compute(b)
            start_out(j, b)
        return 0

    lax.fori_loop(0, SLOTS // 2, outer, 0, unroll=False)
    wait_out(0)
    wait_out(1)


def _sc_aggregate(src_neg, dst_neg):
    mesh = plsc.VectorSubcoreMesh(core_axis_name="c", subcore_axis_name="s")
    f = pl.kernel(
        _sc_agg_body,
        out_type=(jax.ShapeDtypeStruct((NSC, D), jnp.float32),
                  jax.ShapeDtypeStruct((NSC, D), jnp.float32)),
        mesh=mesh,
        scratch_types=[
            pltpu.VMEM((CHUNK_ROWS, D), jnp.float32),
            pltpu.VMEM((CHUNK_ROWS, D), jnp.float32),
            pltpu.VMEM((CH, D), jnp.float32),
            pltpu.VMEM((CH, D), jnp.float32),
            pltpu.SemaphoreType.DMA,
            pltpu.SemaphoreType.DMA,
            pltpu.SemaphoreType.DMA,
            pltpu.SemaphoreType.DMA,
        ],
    )
    return f(src_neg, dst_neg)


def _dot(x, w):
    return jax.lax.dot_general(x, w, (((1,), (0,)), ((), ())),
                               preferred_element_type=jnp.float32,
                               precision=jax.lax.Precision.HIGHEST)


def _tc_fused_body(src_ref, sneg_ref, dst_ref, dneg_ref, w_ref,
                   osrc_ref, odst_ref):
    B = src_ref.shape[0]
    w = w_ref[...]
    sagg = jnp.mean(jnp.reshape(sneg_ref[...], (B, S, D)), axis=1)
    dagg = jnp.mean(jnp.reshape(dneg_ref[...], (B, S, D)), axis=1)
    xs = jnp.concatenate([src_ref[...], sagg], axis=1)
    xd = jnp.concatenate([dst_ref[...], dagg], axis=1)
    osrc_ref[...] = jnp.maximum(_dot(xs, w), 0.0)
    odst_ref[...] = jnp.maximum(_dot(xd, w), 0.0)


def _tc_fused(src, src_neg, dst, dst_neg, w):
    B = 400
    nb = NTC // B
    grid = (nb,)
    row_spec = pl.BlockSpec((B, D), lambda i: (NSC // B + i, 0))
    neg_spec = pl.BlockSpec((B * S, D), lambda i: (NSC // B + i, 0))
    w_spec = pl.BlockSpec((2 * D, DOUT), lambda i: (0, 0))
    out_spec = pl.BlockSpec((B, DOUT), lambda i: (NSC // B + i, 0))
    return pl.pallas_call(
        _tc_fused_body,
        grid=grid,
        in_specs=[row_spec, neg_spec, row_spec, neg_spec, w_spec],
        out_specs=[out_spec, out_spec],
        out_shape=(jax.ShapeDtypeStruct((N, DOUT), jnp.float32),
                   jax.ShapeDtypeStruct((N, DOUT), jnp.float32)),
    )(src, src_neg, dst, dst_neg, w)


def _tc_head_body(src_ref, ssum_ref, dst_ref, dsum_ref, w_ref,
                  _tail_src, _tail_dst, osrc_ref, odst_ref):
    w = w_ref[...]
    inv = jnp.float32(1.0 / S)
    xs = jnp.concatenate([src_ref[...], ssum_ref[...] * inv], axis=1)
    xd = jnp.concatenate([dst_ref[...], dsum_ref[...] * inv], axis=1)
    osrc_ref[...] = jnp.maximum(_dot(xs, w), 0.0)
    odst_ref[...] = jnp.maximum(_dot(xd, w), 0.0)


def _tc_head(src, src_sum, dst, dst_sum, w, tail_src, tail_dst):
    B = 400
    grid = (NSC // B,)
    row_spec = pl.BlockSpec((B, D), lambda i: (i, 0))
    w_spec = pl.BlockSpec((2 * D, DOUT), lambda i: (0, 0))
    any_spec = pl.BlockSpec(memory_space=pl.ANY)
    out_spec = pl.BlockSpec((B, DOUT), lambda i: (i, 0))
    return pl.pallas_call(
        _tc_head_body,
        grid=grid,
        in_specs=[row_spec, row_spec, row_spec, row_spec, w_spec,
                  any_spec, any_spec],
        out_specs=[out_spec, out_spec],
        out_shape=(jax.ShapeDtypeStruct((N, DOUT), jnp.float32),
                   jax.ShapeDtypeStruct((N, DOUT), jnp.float32)),
        input_output_aliases={5: 0, 6: 1},
    )(src, src_sum, dst, dst_sum, w, tail_src, tail_dst)


@jax.jit
def kernel(src, src_neg, dst, dst_neg, w):
    src_sum, dst_sum = _sc_aggregate(src_neg, dst_neg)
    tail_src, tail_dst = _tc_fused(src, src_neg, dst, dst_neg, w)
    out_src, out_dst = _tc_head(src, src_sum, dst, dst_sum, w,
                                tail_src, tail_dst)
    return (out_src, out_dst)
